# Initial kernel scaffold; baseline (speedup 1.0000x reference)
import jax
import jax.numpy as jnp
from jax.experimental import pallas as pl


def kernel(feat, cj, ci, review_feat, prob_w, edge_index):
    raise NotImplementedError("write your pallas kernel here")


# speedup vs baseline: 2.8483x; 2.8483x over previous
import functools

import jax
import jax.numpy as jnp
from jax import lax
from jax.experimental import pallas as pl
from jax.experimental.pallas import tpu as pltpu
from jax.experimental.pallas import tpu_sc as plsc


def _pa_body(rf_ref, w_ref, pa_ref):
    rf = rf_ref[...]
    w = w_ref[...]
    s = jnp.sum(rf * w, axis=1, keepdims=True)
    pa_ref[...] = jax.nn.sigmoid(s)


def _combine_body(p0_ref, p1_ref, ci_ref, o_ref):
    o_ref[...] = (p0_ref[...] + p1_ref[...]) * ci_ref[...]


def _make_sc_kernel(N, E, D):
    NC, NS, L = 2, 16, 16
    NW = NC * NS
    EPT = E // NW
    C = 80
    NCH = EPT // C
    ZR = 128
    NZ = 5
    RPS = ZR * NZ
    Np = RPS * NS
    G = C // L

    mesh = plsc.VectorSubcoreMesh(core_axis_name="c", subcore_axis_name="s")

    @functools.partial(
        pl.kernel,
        out_type=jax.ShapeDtypeStruct((2 * Np, D), jnp.float32),
        mesh=mesh,
        compiler_params=pltpu.CompilerParams(needs_layout_passes=False),
        scratch_types=[
            pltpu.VMEM((C,), jnp.int32),
            pltpu.VMEM((C,), jnp.int32),
            pltpu.VMEM((C,), jnp.float32),
            pltpu.VMEM((C, D), jnp.float32),
            pltpu.VMEM((ZR, D), jnp.float32),
            pltpu.VMEM((N,), jnp.float32),
            pltpu.VMEM_SHARED((Np, D), jnp.float32),
            pltpu.SemaphoreType.DMA,
        ],
    )
    def sck(feat_hbm, cj_hbm, pa_hbm, src_hbm, dst_hbm, out_hbm,
            src_v, dst_v, pa_v, rows_v, zbuf, cj_v, acc, sem):
        cid = lax.axis_index("c")
        sid = lax.axis_index("s")
        wid = sid * NC + cid

        def _zr(r, carry):
            def _zc(c, carry2):
                zbuf[r, pl.ds(c * L, L)] = jnp.zeros((L,), jnp.float32)
                return carry2
            return lax.fori_loop(0, D // L, _zc, carry)
        lax.fori_loop(0, ZR, _zr, 0)
        for k in range(NZ):
            pltpu.sync_copy(zbuf, acc.at[pl.ds(sid * RPS + k * ZR, ZR)])

        pltpu.sync_copy(cj_hbm, cj_v)

        plsc.subcore_barrier()

        def _chunk(ch, carry):
            base = wid * EPT + ch * C
            pltpu.sync_copy(src_hbm.at[pl.ds(base, C)], src_v)
            pltpu.sync_copy(dst_hbm.at[pl.ds(base, C)], dst_v)
            pltpu.sync_copy(pa_hbm.at[pl.ds(base, C)], pa_v)
            pltpu.async_copy(feat_hbm.at[src_v], rows_v, sem).wait()
            for g in range(G):
                src_g = src_v[pl.ds(g * L, L)]
                cj_g = plsc.load_gather(cj_v, [src_g])
                pa_g = pa_v[pl.ds(g * L, L)]
                f = pa_g * cj_g
                for e in range(L):
                    s_vec = jnp.full((L,), f[e], jnp.float32)
                    row = g * L + e
                    for d in range(D // L):
                        rows_v[row, pl.ds(d * L, L)] = (
                            rows_v[row, pl.ds(d * L, L)] * s_vec)
            pltpu.sync_copy(rows_v, acc.at[dst_v], add=True)
            return carry
        lax.fori_loop(0, NCH, _chunk, 0)

        plsc.subcore_barrier()

        for k in range(NZ):
            r0 = sid * RPS + k * ZR
            pltpu.sync_copy(acc.at[pl.ds(r0, ZR)], zbuf)
            pltpu.sync_copy(zbuf, out_hbm.at[pl.ds(cid * Np + r0, ZR)])

    return sck


def kernel(feat, cj, ci, review_feat, prob_w, edge_index):
    N, D = feat.shape
    E, RD = review_feat.shape

    BE = 6400
    pa = pl.pallas_call(
        _pa_body,
        grid=(E // BE,),
        in_specs=[
            pl.BlockSpec((BE, RD), lambda i: (i, 0)),
            pl.BlockSpec((1, RD), lambda i: (0, 0)),
        ],
        out_specs=pl.BlockSpec((BE, 1), lambda i: (i, 0)),
        out_shape=jax.ShapeDtypeStruct((E, 1), jnp.float32),
    )(review_feat, prob_w)

    sck = _make_sc_kernel(N, E, D)
    partial = sck(
        feat,
        cj.reshape(N),
        pa.reshape(E),
        edge_index[0],
        edge_index[1],
    )
    Np = partial.shape[0] // 2
    p0 = partial[:N]
    p1 = partial[Np:Np + N]

    BN = 2000
    NB = N // BN
    out = pl.pallas_call(
        _combine_body,
        grid=(NB,),
        in_specs=[
            pl.BlockSpec((BN, D), lambda i: (i, 0)),
            pl.BlockSpec((BN, D), lambda i: (i, 0)),
            pl.BlockSpec((BN, 1), lambda i: (i, 0)),
        ],
        out_specs=pl.BlockSpec((BN, D), lambda i: (i, 0)),
        out_shape=jax.ShapeDtypeStruct((N, D), jnp.float32),
    )(p0, p1, ci)
    return out


# speedup vs baseline: 4.2885x; 1.5056x over previous
import functools

import jax
import jax.numpy as jnp
from jax import lax
from jax.experimental import pallas as pl
from jax.experimental.pallas import tpu as pltpu
from jax.experimental.pallas import tpu_sc as plsc


def _pa_body(rf_ref, w_ref, pa_ref):
    rf = rf_ref[...]
    w = w_ref[...]
    s = jnp.sum(rf * w, axis=1, keepdims=True)
    pa_ref[...] = jax.nn.sigmoid(s)


def _h_body(f_ref, cj_ref, h_ref):
    h_ref[...] = f_ref[...] * cj_ref[...]


def _combine_body(p0_ref, p1_ref, ci_ref, o_ref):
    o_ref[...] = (p0_ref[...] + p1_ref[...]) * ci_ref[...]


def _make_sc_kernel(N, E, D):
    NC, NS, L = 2, 16, 16
    NW = NC * NS
    EPT = E // NW
    C = 80
    NCH = EPT // C
    G = C // L
    SR = 80
    NSLAB = N // SR
    KMAX = -(-NSLAB // NS)

    mesh = plsc.VectorSubcoreMesh(core_axis_name="c", subcore_axis_name="s")

    @functools.partial(
        pl.kernel,
        out_type=jax.ShapeDtypeStruct((2 * N, D), jnp.float32),
        mesh=mesh,
        compiler_params=pltpu.CompilerParams(needs_layout_passes=False),
        scratch_types=[
            pltpu.VMEM((EPT,), jnp.int32),
            pltpu.VMEM((EPT,), jnp.float32),
            pltpu.VMEM((C,), jnp.int32),
            pltpu.VMEM((C,), jnp.int32),
            pltpu.VMEM((C, D), jnp.float32),
            pltpu.VMEM((C, D), jnp.float32),
            pltpu.VMEM_SHARED((N, D), jnp.float32),
            pltpu.SemaphoreType.DMA,
            pltpu.SemaphoreType.DMA,
            pltpu.SemaphoreType.DMA,
            pltpu.SemaphoreType.DMA,
        ],
    )
    def sck(h_hbm, pa_hbm, src_hbm, dst_hbm, out_hbm,
            src_v, pa_v, dstc0, dstc1, rows0, rows1, acc,
            semg0, semg1, semd0, semd1):
        cid = lax.axis_index("c")
        sid = lax.axis_index("s")
        wid = sid * NC + cid
        ebase = wid * EPT

        pltpu.sync_copy(src_hbm.at[pl.ds(ebase, EPT)], src_v)
        pltpu.sync_copy(pa_hbm.at[pl.ds(ebase, EPT)], pa_v)

        def _zr(r, carry):
            for d in range(D // L):
                rows0[r, pl.ds(d * L, L)] = jnp.zeros((L,), jnp.float32)
            return carry
        lax.fori_loop(0, C, _zr, 0)
        for k in range(KMAX):
            slab = sid + NS * k
            if (k + 1) * NS <= NSLAB:
                pltpu.sync_copy(rows0, acc.at[pl.ds(slab * SR, SR)])
            else:
                @pl.when(slab < NSLAB)
                def _():
                    pltpu.sync_copy(rows0, acc.at[pl.ds(slab * SR, SR)])

        plsc.subcore_barrier()

        def _start_gather(ch, rows, sem):
            pltpu.async_copy(
                h_hbm.at[src_v.at[pl.ds(ch * C, C)]], rows, sem)

        def _wait_gather(ch, rows, sem):
            pltpu.make_async_copy(
                h_hbm.at[src_v.at[pl.ds(ch * C, C)]], rows, sem).wait()

        def _start_dst(ch, dstc, sem):
            pltpu.async_copy(dst_hbm.at[pl.ds(ebase + ch * C, C)], dstc, sem)

        def _wait_dst(ch, dstc, sem):
            pltpu.make_async_copy(
                dst_hbm.at[pl.ds(ebase + ch * C, C)], dstc, sem).wait()

        def _scale(ch, rows):
            def _g(g, carry):
                f = pa_v[pl.ds(ch * C + g * L, L)]
                for e in range(L):
                    s_vec = jnp.full((L,), f[e], jnp.float32)
                    row = g * L + e
                    for d in range(D // L):
                        rows[row, pl.ds(d * L, L)] = (
                            rows[row, pl.ds(d * L, L)] * s_vec)
                return carry
            lax.fori_loop(0, G, _g, 0)

        def _scatter(dstc, rows):
            pltpu.sync_copy(rows, acc.at[dstc], add=True)

        _start_gather(0, rows0, semg0)
        _start_dst(0, dstc0, semd0)

        def _pair(p, carry):
            ch = p * 2
            _wait_gather(ch, rows0, semg0)
            _start_gather(ch + 1, rows1, semg1)
            _start_dst(ch + 1, dstc1, semd1)
            _scale(ch, rows0)
            _wait_dst(ch, dstc0, semd0)
            _scatter(dstc0, rows0)
            _start_dst(ch + 2, dstc0, semd0)
            _wait_gather(ch + 1, rows1, semg1)
            _start_gather(ch + 2, rows0, semg0)
            _scale(ch + 1, rows1)
            _wait_dst(ch + 1, dstc1, semd1)
            _scatter(dstc1, rows1)
            return carry
        lax.fori_loop(0, (NCH - 1) // 2, _pair, 0)

        _wait_gather(NCH - 1, rows0, semg0)
        _scale(NCH - 1, rows0)
        _wait_dst(NCH - 1, dstc0, semd0)
        _scatter(dstc0, rows0)

        plsc.subcore_barrier()

        for k in range(KMAX):
            slab = sid + NS * k
            if (k + 1) * NS <= NSLAB:
                pltpu.sync_copy(acc.at[pl.ds(slab * SR, SR)], rows0)
                pltpu.sync_copy(
                    rows0, out_hbm.at[pl.ds(cid * N + slab * SR, SR)])
            else:
                @pl.when(slab < NSLAB)
                def _():
                    pltpu.sync_copy(acc.at[pl.ds(slab * SR, SR)], rows0)
                    pltpu.sync_copy(
                        rows0, out_hbm.at[pl.ds(cid * N + slab * SR, SR)])

    return sck


def kernel(feat, cj, ci, review_feat, prob_w, edge_index):
    N, D = feat.shape
    E, RD = review_feat.shape

    BE = 6400
    pa = pl.pallas_call(
        _pa_body,
        grid=(E // BE,),
        in_specs=[
            pl.BlockSpec((BE, RD), lambda i: (i, 0)),
            pl.BlockSpec((1, RD), lambda i: (0, 0)),
        ],
        out_specs=pl.BlockSpec((BE, 1), lambda i: (i, 0)),
        out_shape=jax.ShapeDtypeStruct((E, 1), jnp.float32),
    )(review_feat, prob_w)

    BN = 2000
    h = pl.pallas_call(
        _h_body,
        grid=(N // BN,),
        in_specs=[
            pl.BlockSpec((BN, D), lambda i: (i, 0)),
            pl.BlockSpec((BN, 1), lambda i: (i, 0)),
        ],
        out_specs=pl.BlockSpec((BN, D), lambda i: (i, 0)),
        out_shape=jax.ShapeDtypeStruct((N, D), jnp.float32),
    )(feat, cj)

    sck = _make_sc_kernel(N, E, D)
    partial = sck(
        h,
        pa.reshape(E),
        edge_index[0],
        edge_index[1],
    )
    p0 = partial[:N]
    p1 = partial[N:]

    out = pl.pallas_call(
        _combine_body,
        grid=(N // BN,),
        in_specs=[
            pl.BlockSpec((BN, D), lambda i: (i, 0)),
            pl.BlockSpec((BN, D), lambda i: (i, 0)),
            pl.BlockSpec((BN, 1), lambda i: (i, 0)),
        ],
        out_specs=pl.BlockSpec((BN, D), lambda i: (i, 0)),
        out_shape=jax.ShapeDtypeStruct((N, D), jnp.float32),
    )(p0, p1, ci)
    return out


# speedup vs baseline: 7.5022x; 1.7494x over previous
import functools

import jax
import jax.numpy as jnp
from jax import lax
from jax.experimental import pallas as pl
from jax.experimental.pallas import tpu as pltpu
from jax.experimental.pallas import tpu_sc as plsc


def _pa_body(rf_ref, w_ref, pa_ref):
    rf = rf_ref[...]
    w = w_ref[...]
    s = jnp.sum(rf * w, axis=0, keepdims=True)
    pa_ref[...] = jax.nn.sigmoid(s)


def _h_body(f_ref, cj_ref, h_ref):
    h_ref[...] = f_ref[...] * cj_ref[...]


def _combine_body(p0_ref, p1_ref, ci_ref, o_ref):
    o_ref[...] = (p0_ref[...] + p1_ref[...]) * ci_ref[...]


def _make_sc_kernel(N, E, D):
    NC, NS, L = 2, 16, 16
    NW = NC * NS
    EPT = E // NW
    C = 80
    NCH = EPT // C
    G = C // L
    SR = 80
    NSLAB = N // SR
    KMAX = -(-NSLAB // NS)

    mesh = plsc.VectorSubcoreMesh(core_axis_name="c", subcore_axis_name="s")

    @functools.partial(
        pl.kernel,
        out_type=jax.ShapeDtypeStruct((2 * N, D), jnp.float32),
        mesh=mesh,
        compiler_params=pltpu.CompilerParams(needs_layout_passes=False),
        scratch_types=[
            pltpu.VMEM((EPT,), jnp.int32),
            pltpu.VMEM((EPT,), jnp.float32),
            pltpu.VMEM((C,), jnp.int32),
            pltpu.VMEM((C,), jnp.int32),
            pltpu.VMEM((C, D), jnp.float32),
            pltpu.VMEM((C, D), jnp.float32),
            pltpu.VMEM_SHARED((N, D), jnp.float32),
            pltpu.SemaphoreType.DMA,
            pltpu.SemaphoreType.DMA,
            pltpu.SemaphoreType.DMA,
            pltpu.SemaphoreType.DMA,
        ],
    )
    def sck(h_hbm, pa_hbm, src_hbm, dst_hbm, out_hbm,
            src_v, pa_v, dstc0, dstc1, rows0, rows1, acc,
            semg0, semg1, semd0, semd1):
        cid = lax.axis_index("c")
        sid = lax.axis_index("s")
        wid = sid * NC + cid
        ebase = wid * EPT

        pltpu.sync_copy(src_hbm.at[pl.ds(ebase, EPT)], src_v)
        pltpu.sync_copy(pa_hbm.at[pl.ds(ebase, EPT)], pa_v)

        def _zr(r, carry):
            for d in range(D // L):
                rows0[r, pl.ds(d * L, L)] = jnp.zeros((L,), jnp.float32)
            return carry
        lax.fori_loop(0, C, _zr, 0)
        for k in range(KMAX):
            slab = sid + NS * k
            if (k + 1) * NS <= NSLAB:
                pltpu.sync_copy(rows0, acc.at[pl.ds(slab * SR, SR)])
            else:
                @pl.when(slab < NSLAB)
                def _():
                    pltpu.sync_copy(rows0, acc.at[pl.ds(slab * SR, SR)])

        plsc.subcore_barrier()

        def _start_gather(ch, rows, sem):
            pltpu.async_copy(
                h_hbm.at[src_v.at[pl.ds(ch * C, C)]], rows, sem)

        def _wait_gather(ch, rows, sem):
            pltpu.make_async_copy(
                h_hbm.at[src_v.at[pl.ds(ch * C, C)]], rows, sem).wait()

        def _start_dst(ch, dstc, sem):
            pltpu.async_copy(dst_hbm.at[pl.ds(ebase + ch * C, C)], dstc, sem)

        def _wait_dst(ch, dstc, sem):
            pltpu.make_async_copy(
                dst_hbm.at[pl.ds(ebase + ch * C, C)], dstc, sem).wait()

        def _scale(ch, rows):
            def _g(g, carry):
                f = pa_v[pl.ds(ch * C + g * L, L)]
                for e in range(L):
                    s_vec = jnp.full((L,), f[e], jnp.float32)
                    row = g * L + e
                    for d in range(D // L):
                        rows[row, pl.ds(d * L, L)] = (
                            rows[row, pl.ds(d * L, L)] * s_vec)
                return carry
            lax.fori_loop(0, G, _g, 0)

        def _scatter(dstc, rows):
            pltpu.sync_copy(rows, acc.at[dstc], add=True)

        _start_gather(0, rows0, semg0)
        _start_dst(0, dstc0, semd0)

        def _pair(p, carry):
            ch = p * 2
            _wait_gather(ch, rows0, semg0)
            _start_gather(ch + 1, rows1, semg1)
            _start_dst(ch + 1, dstc1, semd1)
            _scale(ch, rows0)
            _wait_dst(ch, dstc0, semd0)
            _scatter(dstc0, rows0)
            _start_dst(ch + 2, dstc0, semd0)
            _wait_gather(ch + 1, rows1, semg1)
            _start_gather(ch + 2, rows0, semg0)
            _scale(ch + 1, rows1)
            _wait_dst(ch + 1, dstc1, semd1)
            _scatter(dstc1, rows1)
            return carry
        lax.fori_loop(0, (NCH - 1) // 2, _pair, 0)

        _wait_gather(NCH - 1, rows0, semg0)
        _scale(NCH - 1, rows0)
        _wait_dst(NCH - 1, dstc0, semd0)
        _scatter(dstc0, rows0)

        plsc.subcore_barrier()

        for k in range(KMAX):
            slab = sid + NS * k
            if (k + 1) * NS <= NSLAB:
                pltpu.sync_copy(acc.at[pl.ds(slab * SR, SR)], rows0)
                pltpu.sync_copy(
                    rows0, out_hbm.at[pl.ds(cid * N + slab * SR, SR)])
            else:
                @pl.when(slab < NSLAB)
                def _():
                    pltpu.sync_copy(acc.at[pl.ds(slab * SR, SR)], rows0)
                    pltpu.sync_copy(
                        rows0, out_hbm.at[pl.ds(cid * N + slab * SR, SR)])

    return sck


def kernel(feat, cj, ci, review_feat, prob_w, edge_index):
    N, D = feat.shape
    E, RD = review_feat.shape

    BE = 6400
    pa = pl.pallas_call(
        _pa_body,
        grid=(E // BE,),
        in_specs=[
            pl.BlockSpec((RD, BE), lambda i: (0, i)),
            pl.BlockSpec((RD, 1), lambda i: (0, 0)),
        ],
        out_specs=pl.BlockSpec((1, BE), lambda i: (0, i)),
        out_shape=jax.ShapeDtypeStruct((1, E), jnp.float32),
    )(review_feat.T, prob_w.T)

    BN = 2000
    h = pl.pallas_call(
        _h_body,
        grid=(N // BN,),
        in_specs=[
            pl.BlockSpec((BN, D), lambda i: (i, 0)),
            pl.BlockSpec((BN, 1), lambda i: (i, 0)),
        ],
        out_specs=pl.BlockSpec((BN, D), lambda i: (i, 0)),
        out_shape=jax.ShapeDtypeStruct((N, D), jnp.float32),
    )(feat, cj)

    sck = _make_sc_kernel(N, E, D)
    partial = sck(
        h,
        pa.reshape(E),
        edge_index[0],
        edge_index[1],
    )
    p0 = partial[:N]
    p1 = partial[N:]

    out = pl.pallas_call(
        _combine_body,
        grid=(N // BN,),
        in_specs=[
            pl.BlockSpec((BN, D), lambda i: (i, 0)),
            pl.BlockSpec((BN, D), lambda i: (i, 0)),
            pl.BlockSpec((BN, 1), lambda i: (i, 0)),
        ],
        out_specs=pl.BlockSpec((BN, D), lambda i: (i, 0)),
        out_shape=jax.ShapeDtypeStruct((N, D), jnp.float32),
    )(p0, p1, ci)
    return out


# speedup vs baseline: 8.5971x; 1.1460x over previous
import functools

import jax
import jax.numpy as jnp
from jax import lax
from jax.experimental import pallas as pl
from jax.experimental.pallas import tpu as pltpu
from jax.experimental.pallas import tpu_sc as plsc


def _pa_body(rf_ref, w_ref, pa_ref):
    rf = rf_ref[...]
    w = w_ref[...]
    s = jnp.sum(rf * w, axis=0, keepdims=True)
    pa_ref[...] = jax.nn.sigmoid(s)


def _h_body(f_ref, cj_ref, h_ref):
    h_ref[...] = f_ref[...] * cj_ref[...]


def _combine_body(p0_ref, p1_ref, ci_ref, o_ref):
    o_ref[...] = (p0_ref[...] + p1_ref[...]) * ci_ref[...]


def _make_sc_kernel(N, E, D):
    NC, NS, L = 2, 16, 16
    NW = NC * NS
    EPT = E // NW
    C = 80
    NCH = EPT // C
    G = C // L
    SR = 80
    NSLAB = N // SR
    KMAX = -(-NSLAB // NS)

    mesh = plsc.VectorSubcoreMesh(core_axis_name="c", subcore_axis_name="s")

    @functools.partial(
        pl.kernel,
        out_type=jax.ShapeDtypeStruct((2 * N, D), jnp.float32),
        mesh=mesh,
        compiler_params=pltpu.CompilerParams(needs_layout_passes=False),
        scratch_types=[
            pltpu.VMEM((EPT,), jnp.int32),
            [pltpu.VMEM((C,), jnp.float32) for _ in range(3)],
            [pltpu.VMEM((C,), jnp.int32) for _ in range(3)],
            [pltpu.VMEM((C, D), jnp.float32) for _ in range(3)],
            pltpu.VMEM_SHARED((N, D), jnp.float32),
            [pltpu.SemaphoreType.DMA for _ in range(3)],
            [pltpu.SemaphoreType.DMA for _ in range(3)],
            [pltpu.SemaphoreType.DMA for _ in range(3)],
            [pltpu.SemaphoreType.DMA for _ in range(3)],
        ],
    )
    def sck(h_hbm, pa_hbm, src_hbm, dst_hbm, out_hbm,
            src_v, pac, dstc, rows, acc, semg, semd, semp, sems):
        cid = lax.axis_index("c")
        sid = lax.axis_index("s")
        wid = sid * NC + cid
        ebase = wid * EPT

        pltpu.sync_copy(src_hbm.at[pl.ds(ebase, EPT)], src_v)

        def _zr(r, carry):
            for d in range(D // L):
                rows[0][r, pl.ds(d * L, L)] = jnp.zeros((L,), jnp.float32)
            return carry
        lax.fori_loop(0, C, _zr, 0)
        for k in range(KMAX):
            slab = sid + NS * k
            if (k + 1) * NS <= NSLAB:
                pltpu.sync_copy(rows[0], acc.at[pl.ds(slab * SR, SR)])
            else:
                @pl.when(slab < NSLAB)
                def _():
                    pltpu.sync_copy(rows[0], acc.at[pl.ds(slab * SR, SR)])

        plsc.subcore_barrier()

        def _start_gather(ch, b):
            pltpu.async_copy(
                h_hbm.at[src_v.at[pl.ds(ch * C, C)]], rows[b], semg[b])

        def _wait_gather(ch, b):
            pltpu.make_async_copy(
                h_hbm.at[src_v.at[pl.ds(ch * C, C)]], rows[b], semg[b]).wait()

        def _start_dst(ch, b):
            pltpu.async_copy(
                dst_hbm.at[pl.ds(ebase + ch * C, C)], dstc[b], semd[b])
            pltpu.async_copy(
                pa_hbm.at[pl.ds(ebase + ch * C, C)], pac[b], semp[b])

        def _wait_dst(ch, b):
            pltpu.make_async_copy(
                dst_hbm.at[pl.ds(ebase + ch * C, C)], dstc[b], semd[b]).wait()

        def _wait_pa(ch, b):
            pltpu.make_async_copy(
                pa_hbm.at[pl.ds(ebase + ch * C, C)], pac[b], semp[b]).wait()

        def _scale(ch, b):
            def _g(g, carry):
                f = pac[b][pl.ds(g * L, L)]
                for e in range(L):
                    s_vec = jnp.full((L,), f[e], jnp.float32)
                    row = g * L + e
                    for d in range(D // L):
                        rows[b][row, pl.ds(d * L, L)] = (
                            rows[b][row, pl.ds(d * L, L)] * s_vec)
                return carry
            lax.fori_loop(0, G, _g, 0)

        def _start_scatter(b):
            pltpu.async_copy(rows[b], acc.at[dstc[b]], sems[b], add=True)

        def _wait_scatter(b):
            pltpu.make_async_copy(rows[b], acc.at[dstc[b]], sems[b]).wait()

        def _sub(ch, b):
            _wait_scatter((b + 1) % 3)
            _start_gather(ch + 1, (b + 1) % 3)
            _start_dst(ch + 1, (b + 1) % 3)
            _wait_gather(ch, b)
            _wait_pa(ch, b)
            _scale(ch, b)
            _wait_dst(ch, b)
            _start_scatter(b)

        _start_gather(0, 0)
        _start_dst(0, 0)
        _start_gather(1, 1)
        _start_dst(1, 1)
        _wait_gather(0, 0)
        _wait_pa(0, 0)
        _scale(0, 0)
        _wait_dst(0, 0)
        _start_scatter(0)
        _start_gather(2, 2)
        _start_dst(2, 2)
        _wait_gather(1, 1)
        _wait_pa(1, 1)
        _scale(1, 1)
        _wait_dst(1, 1)
        _start_scatter(1)

        def _triple(t, carry):
            ch = 3 * t + 2
            _sub(ch, 2)
            _sub(ch + 1, 0)
            _sub(ch + 2, 1)
            return carry
        NT = (NCH - 5) // 3
        lax.fori_loop(0, NT, _triple, 0)

        c0 = NCH - 3
        _sub_last3 = [(c0, 2), (c0 + 1, 0)]
        for ch, b in _sub_last3:
            _wait_scatter((b + 1) % 3)
            _start_gather(ch + 1, (b + 1) % 3)
            _start_dst(ch + 1, (b + 1) % 3)
            _wait_gather(ch, b)
            _wait_pa(ch, b)
            _scale(ch, b)
            _wait_dst(ch, b)
            _start_scatter(b)
        _wait_scatter(2)
        _wait_gather(NCH - 1, 1)
        _wait_pa(NCH - 1, 1)
        _scale(NCH - 1, 1)
        _wait_dst(NCH - 1, 1)
        _start_scatter(1)
        _wait_scatter(0)
        _wait_scatter(1)

        plsc.subcore_barrier()

        for k in range(KMAX):
            slab = sid + NS * k
            if (k + 1) * NS <= NSLAB:
                pltpu.sync_copy(acc.at[pl.ds(slab * SR, SR)], rows[0])
                pltpu.sync_copy(
                    rows[0], out_hbm.at[pl.ds(cid * N + slab * SR, SR)])
            else:
                @pl.when(slab < NSLAB)
                def _():
                    pltpu.sync_copy(acc.at[pl.ds(slab * SR, SR)], rows[0])
                    pltpu.sync_copy(
                        rows[0], out_hbm.at[pl.ds(cid * N + slab * SR, SR)])

    return sck


def kernel(feat, cj, ci, review_feat, prob_w, edge_index):
    N, D = feat.shape
    E, RD = review_feat.shape

    BE = 6400
    pa = pl.pallas_call(
        _pa_body,
        grid=(E // BE,),
        in_specs=[
            pl.BlockSpec((RD, BE), lambda i: (0, i)),
            pl.BlockSpec((RD, 1), lambda i: (0, 0)),
        ],
        out_specs=pl.BlockSpec((1, BE), lambda i: (0, i)),
        out_shape=jax.ShapeDtypeStruct((1, E), jnp.float32),
    )(review_feat.T, prob_w.T)

    BN = 2000
    h = pl.pallas_call(
        _h_body,
        grid=(N // BN,),
        in_specs=[
            pl.BlockSpec((BN, D), lambda i: (i, 0)),
            pl.BlockSpec((BN, 1), lambda i: (i, 0)),
        ],
        out_specs=pl.BlockSpec((BN, D), lambda i: (i, 0)),
        out_shape=jax.ShapeDtypeStruct((N, D), jnp.float32),
    )(feat, cj)

    sck = _make_sc_kernel(N, E, D)
    partial = sck(
        h,
        pa.reshape(E),
        edge_index[0],
        edge_index[1],
    )
    p0 = partial[:N]
    p1 = partial[N:]

    out = pl.pallas_call(
        _combine_body,
        grid=(N // BN,),
        in_specs=[
            pl.BlockSpec((BN, D), lambda i: (i, 0)),
            pl.BlockSpec((BN, D), lambda i: (i, 0)),
            pl.BlockSpec((BN, 1), lambda i: (i, 0)),
        ],
        out_specs=pl.BlockSpec((BN, D), lambda i: (i, 0)),
        out_shape=jax.ShapeDtypeStruct((N, D), jnp.float32),
    )(p0, p1, ci)
    return out


# speedup vs baseline: 8.8064x; 1.0243x over previous
import functools

import jax
import jax.numpy as jnp
from jax import lax
from jax.experimental import pallas as pl
from jax.experimental.pallas import tpu as pltpu
from jax.experimental.pallas import tpu_sc as plsc


def _pa_body(rf_ref, w_ref, pa_ref):
    rf = rf_ref[...]
    w = w_ref[...]
    s = jnp.sum(rf * w, axis=0, keepdims=True)
    pa_ref[...] = jax.nn.sigmoid(s)


def _h_body(f_ref, cj_ref, h_ref):
    h_ref[...] = f_ref[...] * cj_ref[...]


def _combine_body(p0_ref, p1_ref, ci_ref, o_ref):
    o_ref[...] = (p0_ref[...] + p1_ref[...]) * ci_ref[...]


def _make_sc_kernel(N, E, D):
    NC, NS, L = 2, 16, 16
    NW = NC * NS
    EPT = E // NW
    C = 80
    NCH = EPT // C
    G = C // L
    SR = 80
    NSLAB = N // SR
    KMAX = -(-NSLAB // NS)

    mesh = plsc.VectorSubcoreMesh(core_axis_name="c", subcore_axis_name="s")

    @functools.partial(
        pl.kernel,
        out_type=jax.ShapeDtypeStruct((2 * N, D), jnp.float32),
        mesh=mesh,
        compiler_params=pltpu.CompilerParams(needs_layout_passes=False),
        scratch_types=[
            [pltpu.VMEM((C,), jnp.int32) for _ in range(4)],
            [pltpu.VMEM((C,), jnp.float32) for _ in range(4)],
            [pltpu.VMEM((C,), jnp.int32) for _ in range(4)],
            [pltpu.VMEM((C, D), jnp.float32) for _ in range(4)],
            pltpu.VMEM_SHARED((N, D), jnp.float32),
            [pltpu.SemaphoreType.DMA for _ in range(4)],
            [pltpu.SemaphoreType.DMA for _ in range(4)],
            [pltpu.SemaphoreType.DMA for _ in range(4)],
            [pltpu.SemaphoreType.DMA for _ in range(4)],
            [pltpu.SemaphoreType.DMA for _ in range(4)],
        ],
    )
    def sck(h_hbm, pa_hbm, src_hbm, dst_hbm, out_hbm,
            srcc, pac, dstc, rows, acc, semg, semsrc, sempa, semd, sems):
        cid = lax.axis_index("c")
        sid = lax.axis_index("s")
        wid = sid * NC + cid
        ebase = wid * EPT

        def _zr(r, carry):
            for d in range(D // L):
                rows[0][r, pl.ds(d * L, L)] = jnp.zeros((L,), jnp.float32)
            return carry
        lax.fori_loop(0, C, _zr, 0)
        for k in range(KMAX):
            slab = sid + NS * k
            if (k + 1) * NS <= NSLAB:
                pltpu.sync_copy(rows[0], acc.at[pl.ds(slab * SR, SR)])
            else:
                @pl.when(slab < NSLAB)
                def _():
                    pltpu.sync_copy(rows[0], acc.at[pl.ds(slab * SR, SR)])

        plsc.subcore_barrier()

        def _start_gather(b):
            pltpu.async_copy(h_hbm.at[srcc[b]], rows[b], semg[b])

        def _wait_gather(b):
            pltpu.make_async_copy(h_hbm.at[srcc[b]], rows[b], semg[b]).wait()

        def _start_srcpa(ch, b):
            pltpu.async_copy(
                src_hbm.at[pl.ds(ebase + ch * C, C)], srcc[b], semsrc[b])
            pltpu.async_copy(
                pa_hbm.at[pl.ds(ebase + ch * C, C)], pac[b], sempa[b])

        def _wait_srcpa(ch, b):
            pltpu.make_async_copy(
                src_hbm.at[pl.ds(ebase + ch * C, C)], srcc[b], semsrc[b]).wait()
            pltpu.make_async_copy(
                pa_hbm.at[pl.ds(ebase + ch * C, C)], pac[b], sempa[b]).wait()

        def _start_dst(ch, b):
            pltpu.async_copy(
                dst_hbm.at[pl.ds(ebase + ch * C, C)], dstc[b], semd[b])

        def _wait_dst(ch, b):
            pltpu.make_async_copy(
                dst_hbm.at[pl.ds(ebase + ch * C, C)], dstc[b], semd[b]).wait()

        def _scale(ch, b):
            def _g(g, carry):
                f = pac[b][pl.ds(g * L, L)]
                for e in range(L):
                    s_vec = jnp.full((L,), f[e], jnp.float32)
                    row = g * L + e
                    for d in range(D // L):
                        rows[b][row, pl.ds(d * L, L)] = (
                            rows[b][row, pl.ds(d * L, L)] * s_vec)
                return carry
            lax.fori_loop(0, G, _g, 0)

        def _start_scatter(b):
            pltpu.async_copy(rows[b], acc.at[dstc[b]], sems[b], add=True)

        def _wait_scatter(b):
            pltpu.make_async_copy(rows[b], acc.at[dstc[b]], sems[b]).wait()

        def _clamp(ch):
            return jnp.minimum(ch, NCH - 1)

        def _sub(ch, b):
            b2 = (b + 2) % 4
            b3 = (b + 3) % 4
            _wait_scatter(b2)
            _start_dst(_clamp(ch + 2), b2)
            _start_srcpa(_clamp(ch + 3), b3)
            _wait_srcpa(_clamp(ch + 2), b2)
            _start_gather(b2)
            _wait_gather(b)
            _scale(ch, b)
            _wait_dst(ch, b)
            _start_scatter(b)

        _start_srcpa(0, 0)
        _start_srcpa(1, 1)
        _start_srcpa(2, 2)
        _start_dst(0, 0)
        _start_dst(1, 1)
        _wait_srcpa(0, 0)
        _start_gather(0)
        _wait_srcpa(1, 1)
        _start_gather(1)
        _start_dst(2, 2)
        _start_srcpa(3, 3)
        _wait_srcpa(2, 2)
        _start_gather(2)
        _wait_gather(0)
        _scale(0, 0)
        _wait_dst(0, 0)
        _start_scatter(0)
        _start_dst(3, 3)
        _start_srcpa(4, 0)
        _wait_srcpa(3, 3)
        _start_gather(3)
        _wait_gather(1)
        _scale(1, 1)
        _wait_dst(1, 1)
        _start_scatter(1)

        def _quad(t, carry):
            ch = 4 * t + 2
            _sub(ch, 2)
            _sub(ch + 1, 3)
            _sub(ch + 2, 0)
            _sub(ch + 3, 1)
            return carry
        NT = (NCH - 3 - 2) // 4
        lax.fori_loop(0, NT, _quad, 0)

        _sub(NCH - 3, 2)
        _sub(NCH - 2, 3)
        _sub(NCH - 1, 0)

        _wait_scatter(3)
        _wait_scatter(0)
        _wait_gather(1)
        _wait_gather(2)
        _wait_srcpa(NCH - 1, 3)
        _wait_dst(NCH - 1, 1)
        _wait_dst(NCH - 1, 2)

        plsc.subcore_barrier()

        for k in range(KMAX):
            slab = sid + NS * k
            if (k + 1) * NS <= NSLAB:
                pltpu.sync_copy(acc.at[pl.ds(slab * SR, SR)], rows[0])
                pltpu.sync_copy(
                    rows[0], out_hbm.at[pl.ds(cid * N + slab * SR, SR)])
            else:
                @pl.when(slab < NSLAB)
                def _():
                    pltpu.sync_copy(acc.at[pl.ds(slab * SR, SR)], rows[0])
                    pltpu.sync_copy(
                        rows[0], out_hbm.at[pl.ds(cid * N + slab * SR, SR)])

    return sck


def kernel(feat, cj, ci, review_feat, prob_w, edge_index):
    N, D = feat.shape
    E, RD = review_feat.shape

    BE = 6400
    pa = pl.pallas_call(
        _pa_body,
        grid=(E // BE,),
        in_specs=[
            pl.BlockSpec((RD, BE), lambda i: (0, i)),
            pl.BlockSpec((RD, 1), lambda i: (0, 0)),
        ],
        out_specs=pl.BlockSpec((1, BE), lambda i: (0, i)),
        out_shape=jax.ShapeDtypeStruct((1, E), jnp.float32),
    )(review_feat.T, prob_w.T)

    BN = 2000
    h = pl.pallas_call(
        _h_body,
        grid=(N // BN,),
        in_specs=[
            pl.BlockSpec((BN, D), lambda i: (i, 0)),
            pl.BlockSpec((BN, 1), lambda i: (i, 0)),
        ],
        out_specs=pl.BlockSpec((BN, D), lambda i: (i, 0)),
        out_shape=jax.ShapeDtypeStruct((N, D), jnp.float32),
    )(feat, cj)

    sck = _make_sc_kernel(N, E, D)
    partial = sck(
        h,
        pa.reshape(E),
        edge_index[0],
        edge_index[1],
    )
    p0 = partial[:N]
    p1 = partial[N:]

    out = pl.pallas_call(
        _combine_body,
        grid=(N // BN,),
        in_specs=[
            pl.BlockSpec((BN, D), lambda i: (i, 0)),
            pl.BlockSpec((BN, D), lambda i: (i, 0)),
            pl.BlockSpec((BN, 1), lambda i: (i, 0)),
        ],
        out_specs=pl.BlockSpec((BN, D), lambda i: (i, 0)),
        out_shape=jax.ShapeDtypeStruct((N, D), jnp.float32),
    )(p0, p1, ci)
    return out


# speedup vs baseline: 9.1040x; 1.0338x over previous
import functools

import jax
import jax.numpy as jnp
from jax import lax
from jax.experimental import pallas as pl
from jax.experimental.pallas import tpu as pltpu
from jax.experimental.pallas import tpu_sc as plsc


def _pa_body(rf_ref, w_ref, pa_ref):
    rf = rf_ref[...]
    w = w_ref[...]
    s = jnp.sum(rf * w, axis=0, keepdims=True)
    pa_ref[...] = jax.nn.sigmoid(s)


def _h_body(f_ref, cj_ref, h_ref):
    h_ref[...] = f_ref[...] * cj_ref[...]


def _combine_body(p0_ref, p1_ref, ci_ref, o_ref):
    o_ref[...] = (p0_ref[...] + p1_ref[...]) * ci_ref[...]


def _make_sc_kernel(N, E, D):
    NC, NS, L = 2, 16, 16
    NW = NC * NS
    EPT = E // NW
    C = 80
    NCH = EPT // C
    G = C // L
    SR = 80
    NSLAB = N // SR
    KMAX = -(-NSLAB // NS)

    mesh = plsc.VectorSubcoreMesh(core_axis_name="c", subcore_axis_name="s")

    @functools.partial(
        pl.kernel,
        out_type=jax.ShapeDtypeStruct((2 * N, D), jnp.float32),
        mesh=mesh,
        compiler_params=pltpu.CompilerParams(needs_layout_passes=False),
        scratch_types=[
            [pltpu.VMEM((C,), jnp.int32) for _ in range(4)],
            [pltpu.VMEM((C,), jnp.float32) for _ in range(4)],
            [pltpu.VMEM((C,), jnp.int32) for _ in range(4)],
            [pltpu.VMEM((C, D), jnp.float32) for _ in range(4)],
            pltpu.VMEM_SHARED((N, D), jnp.float32),
            [pltpu.SemaphoreType.DMA for _ in range(4)],
            [pltpu.SemaphoreType.DMA for _ in range(4)],
            [pltpu.SemaphoreType.DMA for _ in range(4)],
            [pltpu.SemaphoreType.DMA for _ in range(4)],
            [pltpu.SemaphoreType.DMA for _ in range(4)],
        ],
    )
    def sck(h_hbm, pa_hbm, src_hbm, dst_hbm, out_hbm,
            srcc, pac, dstc, rows, acc, semg, semsrc, sempa, semd, sems):
        cid = lax.axis_index("c")
        sid = lax.axis_index("s")
        wid = sid * NC + cid
        ebase = wid * EPT

        def _zr(r, carry):
            for d in range(D // L):
                rows[0][r, pl.ds(d * L, L)] = jnp.zeros((L,), jnp.float32)
            return carry
        lax.fori_loop(0, C, _zr, 0)
        for k in range(KMAX):
            slab = sid + NS * k
            if (k + 1) * NS <= NSLAB:
                pltpu.sync_copy(rows[0], acc.at[pl.ds(slab * SR, SR)])
            else:
                @pl.when(slab < NSLAB)
                def _():
                    pltpu.sync_copy(rows[0], acc.at[pl.ds(slab * SR, SR)])

        plsc.subcore_barrier()

        def _start_gather(b):
            pltpu.async_copy(h_hbm.at[srcc[b]], rows[b], semg[b])

        def _wait_gather(b):
            pltpu.make_async_copy(h_hbm.at[srcc[b]], rows[b], semg[b]).wait()

        def _start_srcpa(ch, b):
            pltpu.async_copy(
                src_hbm.at[pl.ds(ebase + ch * C, C)], srcc[b], semsrc[b])
            pltpu.async_copy(
                pa_hbm.at[pl.ds(ebase + ch * C, C)], pac[b], sempa[b])

        def _wait_srcpa(ch, b):
            pltpu.make_async_copy(
                src_hbm.at[pl.ds(ebase + ch * C, C)], srcc[b], semsrc[b]).wait()
            pltpu.make_async_copy(
                pa_hbm.at[pl.ds(ebase + ch * C, C)], pac[b], sempa[b]).wait()

        def _start_dst(ch, b):
            pltpu.async_copy(
                dst_hbm.at[pl.ds(ebase + ch * C, C)], dstc[b], semd[b])

        def _wait_dst(ch, b):
            pltpu.make_async_copy(
                dst_hbm.at[pl.ds(ebase + ch * C, C)], dstc[b], semd[b]).wait()

        def _scale(ch, b):
            def _g(g, carry):
                f = pac[b][pl.ds(g * L, L)]
                for e in range(L):
                    s_vec = jnp.full((L,), f[e], jnp.float32)
                    row = g * L + e
                    for d in range(D // L):
                        rows[b][row, pl.ds(d * L, L)] = (
                            rows[b][row, pl.ds(d * L, L)] * s_vec)
                return carry
            lax.fori_loop(0, G, _g, 0)

        def _start_scatter(b):
            pltpu.async_copy(rows[b], acc.at[dstc[b]], sems[b], add=True)

        def _wait_scatter(b):
            pltpu.make_async_copy(rows[b], acc.at[dstc[b]], sems[b]).wait()

        def _clamp(ch):
            return jnp.minimum(ch, NCH - 1)

        def _sub(ch, b):
            b2 = (b + 2) % 4
            b3 = (b + 3) % 4
            _wait_scatter(b2)
            _start_dst(_clamp(ch + 2), b2)
            _start_srcpa(_clamp(ch + 3), b3)
            _wait_srcpa(_clamp(ch + 2), b2)
            _start_gather(b2)
            _wait_gather(b)
            _scale(ch, b)
            _wait_dst(ch, b)
            _start_scatter(b)

        _start_srcpa(0, 0)
        _start_srcpa(1, 1)
        _start_srcpa(2, 2)
        _start_dst(0, 0)
        _start_dst(1, 1)
        _wait_srcpa(0, 0)
        _start_gather(0)
        _wait_srcpa(1, 1)
        _start_gather(1)
        _start_dst(2, 2)
        _start_srcpa(3, 3)
        _wait_srcpa(2, 2)
        _start_gather(2)
        _wait_gather(0)
        _scale(0, 0)
        _wait_dst(0, 0)
        _start_scatter(0)
        _start_dst(3, 3)
        _start_srcpa(4, 0)
        _wait_srcpa(3, 3)
        _start_gather(3)
        _wait_gather(1)
        _scale(1, 1)
        _wait_dst(1, 1)
        _start_scatter(1)

        def _quad(t, carry):
            ch = 4 * t + 2
            _sub(ch, 2)
            _sub(ch + 1, 3)
            _sub(ch + 2, 0)
            _sub(ch + 3, 1)
            return carry
        NT = (NCH - 3 - 2) // 4
        lax.fori_loop(0, NT, _quad, 0)

        _sub(NCH - 3, 2)
        _sub(NCH - 2, 3)
        _sub(NCH - 1, 0)

        _wait_scatter(3)
        _wait_scatter(0)
        _wait_gather(1)
        _wait_gather(2)
        _wait_srcpa(NCH - 1, 3)
        _wait_dst(NCH - 1, 1)
        _wait_dst(NCH - 1, 2)

        plsc.subcore_barrier()

        for k in range(KMAX):
            slab = sid + NS * k
            if (k + 1) * NS <= NSLAB:
                pltpu.sync_copy(acc.at[pl.ds(slab * SR, SR)], rows[0])
                pltpu.sync_copy(
                    rows[0], out_hbm.at[pl.ds(cid * N + slab * SR, SR)])
            else:
                @pl.when(slab < NSLAB)
                def _():
                    pltpu.sync_copy(acc.at[pl.ds(slab * SR, SR)], rows[0])
                    pltpu.sync_copy(
                        rows[0], out_hbm.at[pl.ds(cid * N + slab * SR, SR)])

    return sck


def kernel(feat, cj, ci, review_feat, prob_w, edge_index):
    N, D = feat.shape
    E, RD = review_feat.shape

    BE = 6400
    pa = pl.pallas_call(
        _pa_body,
        grid=(E // BE,),
        in_specs=[
            pl.BlockSpec((RD, BE), lambda i: (0, i)),
            pl.BlockSpec((RD, 1), lambda i: (0, 0)),
        ],
        out_specs=pl.BlockSpec((1, BE), lambda i: (0, i)),
        out_shape=jax.ShapeDtypeStruct((1, E), jnp.float32),
    )(review_feat.T, prob_w.T)

    BN = 2000
    h = pl.pallas_call(
        _h_body,
        grid=(N // BN,),
        in_specs=[
            pl.BlockSpec((BN, D), lambda i: (i, 0)),
            pl.BlockSpec((BN, 1), lambda i: (i, 0)),
        ],
        out_specs=pl.BlockSpec((BN, D), lambda i: (i, 0)),
        out_shape=jax.ShapeDtypeStruct((N, D), jnp.float32),
    )(feat, cj)

    sck = _make_sc_kernel(N, E, D)
    partial = sck(
        h,
        pa.reshape(E),
        edge_index[0],
        edge_index[1],
    )
    NB = N // BN
    out = pl.pallas_call(
        _combine_body,
        grid=(NB,),
        in_specs=[
            pl.BlockSpec((BN, D), lambda i: (i, 0)),
            pl.BlockSpec((BN, D), lambda i: (i + NB, 0)),
            pl.BlockSpec((BN, 1), lambda i: (i, 0)),
        ],
        out_specs=pl.BlockSpec((BN, D), lambda i: (i, 0)),
        out_shape=jax.ShapeDtypeStruct((N, D), jnp.float32),
    )(partial, partial, ci)
    return out


# speedup vs baseline: 9.4131x; 1.0340x over previous
import functools

import jax
import jax.numpy as jnp
from jax import lax
from jax.experimental import pallas as pl
from jax.experimental.pallas import tpu as pltpu
from jax.experimental.pallas import tpu_sc as plsc


def _pa_body(rf_ref, w_ref, pa_ref):
    rf = rf_ref[...]
    w = w_ref[...]
    s = jnp.sum(rf * w, axis=0, keepdims=True)
    pa_ref[...] = jax.nn.sigmoid(s)


def _h_body(f_ref, cj_ref, h_ref):
    h_ref[...] = f_ref[...] * cj_ref[...]


def _combine_body(p0_ref, p1_ref, ci_ref, o_ref):
    o_ref[...] = (p0_ref[...] + p1_ref[...]) * ci_ref[...]


def _make_sc_kernel(N, E, D, nch, origin, load_acc):
    NC, NS, L = 2, 16, 16
    NW = NC * NS
    C = 80
    EPT = nch * C
    G = C // L
    SR = 80
    NSLAB = N // SR
    KMAX = -(-NSLAB // NS)

    mesh = plsc.VectorSubcoreMesh(core_axis_name="c", subcore_axis_name="s")

    def _impl(h_hbm, pa_hbm, src_hbm, dst_hbm, pin_hbm, out_hbm,
              srcc, pac, dstc, rows, acc, semg, semsrc, sempa, semd, sems):
        cid = lax.axis_index("c")
        sid = lax.axis_index("s")
        wid = sid * NC + cid
        ebase = origin + wid * EPT
        pbase = wid * EPT

        def _slabs(fn):
            for k in range(KMAX):
                slab = sid + NS * k
                if (k + 1) * NS <= NSLAB:
                    fn(slab)
                else:
                    @pl.when(slab < NSLAB)
                    def _():
                        fn(slab)

        if load_acc:
            def _load(slab):
                pltpu.sync_copy(
                    pin_hbm.at[pl.ds(cid * N + slab * SR, SR)], rows[0])
                pltpu.sync_copy(rows[0], acc.at[pl.ds(slab * SR, SR)])
            _slabs(_load)
        else:
            def _zr(r, carry):
                for d in range(D // L):
                    rows[0][r, pl.ds(d * L, L)] = jnp.zeros((L,), jnp.float32)
                return carry
            lax.fori_loop(0, C, _zr, 0)

            def _zero(slab):
                pltpu.sync_copy(rows[0], acc.at[pl.ds(slab * SR, SR)])
            _slabs(_zero)

        plsc.subcore_barrier()

        def _start_gather(b):
            pltpu.async_copy(h_hbm.at[srcc[b]], rows[b], semg[b])

        def _wait_gather(b):
            pltpu.make_async_copy(h_hbm.at[srcc[b]], rows[b], semg[b]).wait()

        def _start_srcpa(ch, b):
            pltpu.async_copy(
                src_hbm.at[pl.ds(ebase + ch * C, C)], srcc[b], semsrc[b])
            pltpu.async_copy(
                pa_hbm.at[pl.ds(pbase + ch * C, C)], pac[b], sempa[b])

        def _wait_srcpa(ch, b):
            pltpu.make_async_copy(
                src_hbm.at[pl.ds(ebase + ch * C, C)], srcc[b], semsrc[b]).wait()
            pltpu.make_async_copy(
                pa_hbm.at[pl.ds(pbase + ch * C, C)], pac[b], sempa[b]).wait()

        def _start_dst(ch, b):
            pltpu.async_copy(
                dst_hbm.at[pl.ds(ebase + ch * C, C)], dstc[b], semd[b])

        def _wait_dst(ch, b):
            pltpu.make_async_copy(
                dst_hbm.at[pl.ds(ebase + ch * C, C)], dstc[b], semd[b]).wait()

        def _scale(ch, b):
            def _g(g, carry):
                f = pac[b][pl.ds(g * L, L)]
                for e in range(L):
                    s_vec = jnp.full((L,), f[e], jnp.float32)
                    row = g * L + e
                    for d in range(D // L):
                        rows[b][row, pl.ds(d * L, L)] = (
                            rows[b][row, pl.ds(d * L, L)] * s_vec)
                return carry
            lax.fori_loop(0, G, _g, 0)

        def _start_scatter(b):
            pltpu.async_copy(rows[b], acc.at[dstc[b]], sems[b], add=True)

        def _wait_scatter(b):
            pltpu.make_async_copy(rows[b], acc.at[dstc[b]], sems[b]).wait()

        def _clamp(ch):
            return jnp.minimum(ch, nch - 1)

        def _sub(ch, b):
            b2 = (b + 2) % 4
            b3 = (b + 3) % 4
            _wait_scatter(b2)
            _start_dst(_clamp(ch + 2), b2)
            _start_srcpa(_clamp(ch + 3), b3)
            _wait_srcpa(_clamp(ch + 2), b2)
            _start_gather(b2)
            _wait_gather(b)
            _scale(ch, b)
            _wait_dst(ch, b)
            _start_scatter(b)

        _start_srcpa(0, 0)
        _start_srcpa(1, 1)
        _start_srcpa(2, 2)
        _start_dst(0, 0)
        _start_dst(1, 1)
        _wait_srcpa(0, 0)
        _start_gather(0)
        _wait_srcpa(1, 1)
        _start_gather(1)
        _start_dst(2, 2)
        _start_srcpa(3, 3)
        _wait_srcpa(2, 2)
        _start_gather(2)
        _wait_gather(0)
        _scale(0, 0)
        _wait_dst(0, 0)
        _start_scatter(0)
        _start_dst(3, 3)
        _start_srcpa(4, 0)
        _wait_srcpa(3, 3)
        _start_gather(3)
        _wait_gather(1)
        _scale(1, 1)
        _wait_dst(1, 1)
        _start_scatter(1)

        NT = (nch - 2) // 4

        def _quad(t, carry):
            ch = 4 * t + 2
            _sub(ch, 2)
            _sub(ch + 1, 3)
            _sub(ch + 2, 0)
            _sub(ch + 3, 1)
            return carry
        lax.fori_loop(0, NT, _quad, 0)
        for ch in range(2 + 4 * NT, nch):
            _sub(ch, ch % 4)

        _wait_scatter((nch - 2) % 4)
        _wait_scatter((nch - 1) % 4)
        _wait_gather(nch % 4)
        _wait_gather((nch + 1) % 4)
        _wait_srcpa(nch - 1, (nch + 2) % 4)
        _wait_dst(nch - 1, nch % 4)
        _wait_dst(nch - 1, (nch + 1) % 4)

        plsc.subcore_barrier()

        def _dump(slab):
            pltpu.sync_copy(acc.at[pl.ds(slab * SR, SR)], rows[0])
            pltpu.sync_copy(
                rows[0], out_hbm.at[pl.ds(cid * N + slab * SR, SR)])
        _slabs(_dump)

    kw = dict(
        out_type=jax.ShapeDtypeStruct((2 * N, D), jnp.float32),
        mesh=mesh,
        compiler_params=pltpu.CompilerParams(needs_layout_passes=False),
        scratch_types=[
            [pltpu.VMEM((C,), jnp.int32) for _ in range(4)],
            [pltpu.VMEM((C,), jnp.float32) for _ in range(4)],
            [pltpu.VMEM((C,), jnp.int32) for _ in range(4)],
            [pltpu.VMEM((C, D), jnp.float32) for _ in range(4)],
            pltpu.VMEM_SHARED((N, D), jnp.float32),
            [pltpu.SemaphoreType.DMA for _ in range(4)],
            [pltpu.SemaphoreType.DMA for _ in range(4)],
            [pltpu.SemaphoreType.DMA for _ in range(4)],
            [pltpu.SemaphoreType.DMA for _ in range(4)],
            [pltpu.SemaphoreType.DMA for _ in range(4)],
        ],
    )

    if load_acc:
        @functools.partial(pl.kernel, **kw)
        def sck(h_hbm, pa_hbm, src_hbm, dst_hbm, pin_hbm, out_hbm,
                srcc, pac, dstc, rows, acc, semg, semsrc, sempa, semd, sems):
            _impl(h_hbm, pa_hbm, src_hbm, dst_hbm, pin_hbm, out_hbm,
                  srcc, pac, dstc, rows, acc, semg, semsrc, sempa, semd, sems)
    else:
        @functools.partial(pl.kernel, **kw)
        def sck(h_hbm, pa_hbm, src_hbm, dst_hbm, out_hbm,
                srcc, pac, dstc, rows, acc, semg, semsrc, sempa, semd, sems):
            _impl(h_hbm, pa_hbm, src_hbm, dst_hbm, None, out_hbm,
                  srcc, pac, dstc, rows, acc, semg, semsrc, sempa, semd, sems)
    return sck


def _pa_slice(rfT, prob_wT, E_k, block_off):
    RD = rfT.shape[0]
    BE = 6400
    return pl.pallas_call(
        _pa_body,
        grid=(E_k // BE,),
        in_specs=[
            pl.BlockSpec((RD, BE), lambda i: (0, i + block_off)),
            pl.BlockSpec((RD, 1), lambda i: (0, 0)),
        ],
        out_specs=pl.BlockSpec((1, BE), lambda i: (0, i)),
        out_shape=jax.ShapeDtypeStruct((1, E_k), jnp.float32),
    )(rfT, prob_wT)


def kernel(feat, cj, ci, review_feat, prob_w, edge_index):
    N, D = feat.shape
    E, RD = review_feat.shape

    NW, C = 32, 80
    NCH0 = 60
    E0 = NW * C * NCH0
    NCH1 = (E - E0) // (NW * C)
    E1 = E - E0
    BE = 6400

    rfT = review_feat.T
    wT = prob_w.T
    pa0 = _pa_slice(rfT, wT, E0, 0)
    pa1 = _pa_slice(rfT, wT, E1, E0 // BE)

    BN = 2000
    h = pl.pallas_call(
        _h_body,
        grid=(N // BN,),
        in_specs=[
            pl.BlockSpec((BN, D), lambda i: (i, 0)),
            pl.BlockSpec((BN, 1), lambda i: (i, 0)),
        ],
        out_specs=pl.BlockSpec((BN, D), lambda i: (i, 0)),
        out_shape=jax.ShapeDtypeStruct((N, D), jnp.float32),
    )(feat, cj)

    src = edge_index[0]
    dst = edge_index[1]
    sck0 = _make_sc_kernel(N, E, D, NCH0, 0, False)
    sck1 = _make_sc_kernel(N, E, D, NCH1, E0, True)
    partial0 = sck0(h, pa0.reshape(E0), src, dst)
    partial = sck1(h, pa1.reshape(E1), src, dst, partial0)

    NB = N // BN
    out = pl.pallas_call(
        _combine_body,
        grid=(NB,),
        in_specs=[
            pl.BlockSpec((BN, D), lambda i: (i, 0)),
            pl.BlockSpec((BN, D), lambda i: (i + NB, 0)),
            pl.BlockSpec((BN, 1), lambda i: (i, 0)),
        ],
        out_specs=pl.BlockSpec((BN, D), lambda i: (i, 0)),
        out_shape=jax.ShapeDtypeStruct((N, D), jnp.float32),
    )(partial, partial, ci)
    return out


# speedup vs baseline: 9.7177x; 1.0324x over previous
import functools

import jax
import jax.numpy as jnp
from jax import lax
from jax.experimental import pallas as pl
from jax.experimental.pallas import tpu as pltpu
from jax.experimental.pallas import tpu_sc as plsc


def _pa_body(rf_ref, w_ref, pa_ref):
    rf = rf_ref[...]
    w = w_ref[...]
    s = jnp.sum(rf * w, axis=0, keepdims=True)
    pa_ref[...] = jax.nn.sigmoid(s)


def _h_body(f_ref, cj_ref, h_ref):
    h_ref[...] = f_ref[...] * cj_ref[...]


def _combine_body(p0_ref, p1_ref, ci_ref, o_ref):
    o_ref[...] = (p0_ref[...] + p1_ref[...]) * ci_ref[...]


def _make_sc_kernel(N, E, D, nch, origin, load_acc):
    NC, NS, L = 2, 16, 16
    NW = NC * NS
    C = 80
    EPT = nch * C
    G = C // L
    SR = 80
    NSLAB = N // SR
    KMAX = -(-NSLAB // NS)

    mesh = plsc.VectorSubcoreMesh(core_axis_name="c", subcore_axis_name="s")

    def _impl(h_hbm, pa_hbm, src_hbm, dst_hbm, pin_hbm, out_hbm,
              srcc, pac, dstc, rows, acc, semg, semsrc, sempa, semd, sems):
        cid = lax.axis_index("c")
        sid = lax.axis_index("s")
        wid = sid * NC + cid
        ebase = origin + wid * EPT
        pbase = wid * EPT

        def _slabs(fn):
            for k in range(KMAX):
                slab = sid + NS * k
                if (k + 1) * NS <= NSLAB:
                    fn(slab)
                else:
                    @pl.when(slab < NSLAB)
                    def _():
                        fn(slab)

        if load_acc:
            def _load(slab):
                pltpu.sync_copy(
                    pin_hbm.at[pl.ds(cid * N + slab * SR, SR)], rows[0])
                pltpu.sync_copy(rows[0], acc.at[pl.ds(slab * SR, SR)])
            _slabs(_load)
        else:
            def _zr(r, carry):
                for d in range(D // L):
                    rows[0][r, pl.ds(d * L, L)] = jnp.zeros((L,), jnp.float32)
                return carry
            lax.fori_loop(0, C, _zr, 0)

            def _zero(slab):
                pltpu.sync_copy(rows[0], acc.at[pl.ds(slab * SR, SR)])
            _slabs(_zero)

        plsc.subcore_barrier()

        def _start_gather(b):
            pltpu.async_copy(h_hbm.at[srcc[b]], rows[b], semg[b])

        def _wait_gather(b):
            pltpu.make_async_copy(h_hbm.at[srcc[b]], rows[b], semg[b]).wait()

        def _start_srcpa(ch, b):
            pltpu.async_copy(
                src_hbm.at[pl.ds(ebase + ch * C, C)], srcc[b], semsrc[b])
            pltpu.async_copy(
                pa_hbm.at[pl.ds(pbase + ch * C, C)], pac[b], sempa[b])

        def _wait_srcpa(ch, b):
            pltpu.make_async_copy(
                src_hbm.at[pl.ds(ebase + ch * C, C)], srcc[b], semsrc[b]).wait()
            pltpu.make_async_copy(
                pa_hbm.at[pl.ds(pbase + ch * C, C)], pac[b], sempa[b]).wait()

        def _start_dst(ch, b):
            pltpu.async_copy(
                dst_hbm.at[pl.ds(ebase + ch * C, C)], dstc[b], semd[b])

        def _wait_dst(ch, b):
            pltpu.make_async_copy(
                dst_hbm.at[pl.ds(ebase + ch * C, C)], dstc[b], semd[b]).wait()

        def _scale(ch, b):
            def _g(g, carry):
                f = pac[b][pl.ds(g * L, L)]
                for e in range(L):
                    s_vec = jnp.full((L,), f[e], jnp.float32)
                    row = g * L + e
                    for d in range(D // L):
                        rows[b][row, pl.ds(d * L, L)] = (
                            rows[b][row, pl.ds(d * L, L)] * s_vec)
                return carry
            lax.fori_loop(0, G, _g, 0)

        def _start_scatter(b):
            pltpu.async_copy(rows[b], acc.at[dstc[b]], sems[b], add=True)

        def _wait_scatter(b):
            pltpu.make_async_copy(rows[b], acc.at[dstc[b]], sems[b]).wait()

        def _clamp(ch):
            return jnp.minimum(ch, nch - 1)

        def _sub(ch, b):
            b2 = (b + 2) % 4
            b3 = (b + 3) % 4
            _wait_scatter(b2)
            _start_dst(_clamp(ch + 2), b2)
            _start_srcpa(_clamp(ch + 3), b3)
            _wait_srcpa(_clamp(ch + 2), b2)
            _start_gather(b2)
            _wait_gather(b)
            _scale(ch, b)
            _wait_dst(ch, b)
            _start_scatter(b)

        _start_srcpa(0, 0)
        _start_srcpa(1, 1)
        _start_srcpa(2, 2)
        _start_dst(0, 0)
        _start_dst(1, 1)
        _wait_srcpa(0, 0)
        _start_gather(0)
        _wait_srcpa(1, 1)
        _start_gather(1)
        _start_dst(2, 2)
        _start_srcpa(3, 3)
        _wait_srcpa(2, 2)
        _start_gather(2)
        _wait_gather(0)
        _scale(0, 0)
        _wait_dst(0, 0)
        _start_scatter(0)
        _start_dst(3, 3)
        _start_srcpa(4, 0)
        _wait_srcpa(3, 3)
        _start_gather(3)
        _wait_gather(1)
        _scale(1, 1)
        _wait_dst(1, 1)
        _start_scatter(1)

        NT = (nch - 2) // 4

        def _quad(t, carry):
            ch = 4 * t + 2
            _sub(ch, 2)
            _sub(ch + 1, 3)
            _sub(ch + 2, 0)
            _sub(ch + 3, 1)
            return carry
        lax.fori_loop(0, NT, _quad, 0)
        for ch in range(2 + 4 * NT, nch):
            _sub(ch, ch % 4)

        _wait_scatter((nch - 2) % 4)
        _wait_scatter((nch - 1) % 4)
        _wait_gather(nch % 4)
        _wait_gather((nch + 1) % 4)
        _wait_srcpa(nch - 1, (nch + 2) % 4)
        _wait_dst(nch - 1, nch % 4)
        _wait_dst(nch - 1, (nch + 1) % 4)

        plsc.subcore_barrier()

        def _dump(slab):
            pltpu.sync_copy(acc.at[pl.ds(slab * SR, SR)], rows[0])
            pltpu.sync_copy(
                rows[0], out_hbm.at[pl.ds(cid * N + slab * SR, SR)])
        _slabs(_dump)

    kw = dict(
        out_type=jax.ShapeDtypeStruct((2 * N, D), jnp.float32),
        mesh=mesh,
        compiler_params=pltpu.CompilerParams(needs_layout_passes=False),
        scratch_types=[
            [pltpu.VMEM((C,), jnp.int32) for _ in range(4)],
            [pltpu.VMEM((C,), jnp.float32) for _ in range(4)],
            [pltpu.VMEM((C,), jnp.int32) for _ in range(4)],
            [pltpu.VMEM((C, D), jnp.float32) for _ in range(4)],
            pltpu.VMEM_SHARED((N, D), jnp.float32),
            [pltpu.SemaphoreType.DMA for _ in range(4)],
            [pltpu.SemaphoreType.DMA for _ in range(4)],
            [pltpu.SemaphoreType.DMA for _ in range(4)],
            [pltpu.SemaphoreType.DMA for _ in range(4)],
            [pltpu.SemaphoreType.DMA for _ in range(4)],
        ],
    )

    if load_acc:
        @functools.partial(pl.kernel, **kw)
        def sck(h_hbm, pa_hbm, src_hbm, dst_hbm, pin_hbm, out_hbm,
                srcc, pac, dstc, rows, acc, semg, semsrc, sempa, semd, sems):
            _impl(h_hbm, pa_hbm, src_hbm, dst_hbm, pin_hbm, out_hbm,
                  srcc, pac, dstc, rows, acc, semg, semsrc, sempa, semd, sems)
    else:
        @functools.partial(pl.kernel, **kw)
        def sck(h_hbm, pa_hbm, src_hbm, dst_hbm, out_hbm,
                srcc, pac, dstc, rows, acc, semg, semsrc, sempa, semd, sems):
            _impl(h_hbm, pa_hbm, src_hbm, dst_hbm, None, out_hbm,
                  srcc, pac, dstc, rows, acc, semg, semsrc, sempa, semd, sems)
    return sck


def _pa_slice(rfT, prob_wT, E_k, block_off):
    RD = rfT.shape[0]
    BE = 6400
    return pl.pallas_call(
        _pa_body,
        grid=(E_k // BE,),
        in_specs=[
            pl.BlockSpec((RD, BE), lambda i: (0, i + block_off)),
            pl.BlockSpec((RD, 1), lambda i: (0, 0)),
        ],
        out_specs=pl.BlockSpec((1, BE), lambda i: (0, i)),
        out_shape=jax.ShapeDtypeStruct((1, E_k), jnp.float32),
    )(rfT, prob_wT)


def kernel(feat, cj, ci, review_feat, prob_w, edge_index):
    N, D = feat.shape
    E, RD = review_feat.shape

    NW, C = 32, 80
    NCH0 = 35
    E0 = NW * C * NCH0
    NCH1 = (E - E0) // (NW * C)
    E1 = E - E0
    BE = 6400

    rfT = review_feat.T
    wT = prob_w.T
    pa0 = _pa_slice(rfT, wT, E0, 0)
    pa1 = _pa_slice(rfT, wT, E1, E0 // BE)

    BN = 2000
    h = pl.pallas_call(
        _h_body,
        grid=(N // BN,),
        in_specs=[
            pl.BlockSpec((BN, D), lambda i: (i, 0)),
            pl.BlockSpec((BN, 1), lambda i: (i, 0)),
        ],
        out_specs=pl.BlockSpec((BN, D), lambda i: (i, 0)),
        out_shape=jax.ShapeDtypeStruct((N, D), jnp.float32),
    )(feat, cj)

    src = edge_index[0]
    dst = edge_index[1]
    sck0 = _make_sc_kernel(N, E, D, NCH0, 0, False)
    sck1 = _make_sc_kernel(N, E, D, NCH1, E0, True)
    partial0 = sck0(h, pa0.reshape(E0), src, dst)
    partial = sck1(h, pa1.reshape(E1), src, dst, partial0)

    NB = N // BN
    out = pl.pallas_call(
        _combine_body,
        grid=(NB,),
        in_specs=[
            pl.BlockSpec((BN, D), lambda i: (i, 0)),
            pl.BlockSpec((BN, D), lambda i: (i + NB, 0)),
            pl.BlockSpec((BN, 1), lambda i: (i, 0)),
        ],
        out_specs=pl.BlockSpec((BN, D), lambda i: (i, 0)),
        out_shape=jax.ShapeDtypeStruct((N, D), jnp.float32),
    )(partial, partial, ci)
    return out
